# baseline (speedup 1.0000x reference)
import functools

import jax
import jax.numpy as jnp
from jax import lax
from jax.experimental import pallas as pl
from jax.experimental.pallas import tpu as pltpu
from jax.experimental.pallas import tpu_sc as plsc

N = 4096
G = 4 * N
NNZ = 671088
DIN = 256
B = 32
T = 16

NC = 2
NS = 16
HB = B // NC
CH = 512
DEPTH = 5
NCHUNK = 85
NNZ_T = CH * NCHUNK
NSTR = 1
NSEG = N // NS
GSEG = G // NS
NNZ_PAD = NS * NNZ_T


def _xproj_body(x_ref, w_ref, b_ref, o_ref):
    o_ref[...] = (
        jnp.dot(x_ref[...], w_ref[...], preferred_element_type=jnp.float32)
        + b_ref[...][None, :]
    )


_xproj_call = pl.pallas_call(
    _xproj_body,
    grid=(G // 512,),
    in_specs=[
        pl.BlockSpec((B * T, DIN), lambda g: (0, 0)),
        pl.BlockSpec((DIN, 512), lambda g: (0, g)),
        pl.BlockSpec((512,), lambda g: (g,)),
    ],
    out_specs=pl.BlockSpec((B * T, 512), lambda g: (0, g)),
    out_shape=jax.ShapeDtypeStruct((B * T, G), jnp.float32),
)


def _splat(v, j):
    idx = jnp.full((16, 1), j, dtype=jnp.int32)
    dnums = lax.GatherDimensionNumbers(
        offset_dims=(), collapsed_slice_dims=(0,), start_index_map=(0,)
    )
    return lax.gather(
        v, idx, dnums, slice_sizes=(1,),
        mode=lax.GatherScatterMode.PROMISE_IN_BOUNDS,
    )


def _sigmoid(x):
    return 1.0 / (1.0 + jnp.exp(-x))


def _tanh(x):
    return 2.0 / (1.0 + jnp.exp(-2.0 * x)) - 1.0


def _sc_body(xp_hbm, idx_hbm, vals_hbm, out_hbm, h_hbm,
             ib0, ib1, ib2, ib3, ib4, vb0, vb1, vb2, vb3, vb4,
             gb0, gb1, gb2, gb3, gb4,
             ibuf, fbuf, g2buf, obuf, cbuf, hbuf,
             gates_sh, semI, semG, semS):
    c = lax.axis_index("c")
    s = lax.axis_index("s")
    n0 = s * NSEG
    g0 = s * GSEG
    grp0 = s * (NNZ_T // 128)

    ib = (ib0, ib1, ib2, ib3, ib4)
    vb = (vb0, vb1, vb2, vb3, vb4)
    gb = (gb0, gb1, gb2, gb3, gb4)

    def _fire_idx(k, p):
        pltpu.async_copy(idx_hbm.at[s * NCHUNK + k], ib[p], semI)
        pltpu.async_copy(vals_hbm.at[pl.ds(s * NNZ_T + k * CH, CH)], vb[p], semI)

    def _wait_idx(p):
        pltpu.make_async_copy(idx_hbm.at[0], ib[p], semI).wait()
        pltpu.make_async_copy(vals_hbm.at[pl.ds(0, CH)], vb[p], semI).wait()
        coff = c * N

        def _addoff(q, carry):
            ib[p][0, pl.ds(q * 16, 16)] = ib[p][0, pl.ds(q * 16, 16)] + coff
            return carry

        lax.fori_loop(0, CH // 16, _addoff, 0)

    def _fire_gather(p):
        pltpu.async_copy(h_hbm.at[ib[p].at[0]], gb[p], semG)

    def _wait_gather(p):
        pltpu.make_async_copy(h_hbm.at[ib[p].at[0]], gb[p], semG).wait()

    def _fire_scatter(p):
        pltpu.async_copy(gb[p], gates_sh.at[ib[p].at[1]], semS, add=True)

    def _wait_scatter(p):
        pltpu.make_async_copy(gb[p], gates_sh.at[ib[p].at[1]], semS).wait()

    def _mult(p):
        @plsc.parallel_loop(0, CH // 16, unroll=2)
        def _body(q):
            vv = vb[p][pl.ds(q * 16, 16)]
            base = q * 16
            for j in range(16):
                gb[p][base + j] = gb[p][base + j] * _splat(vv, j)

    def _zero(r, carry):
        hbuf[r] = jnp.zeros((HB,), jnp.float32)
        cbuf[r] = jnp.zeros((HB,), jnp.float32)
        return carry

    lax.fori_loop(0, NSEG, _zero, 0)
    pltpu.sync_copy(hbuf, h_hbm.at[pl.ds(c * N + n0, NSEG)])
    plsc.subcore_barrier()

    def _step(t, carry):
        pltpu.sync_copy(xp_hbm.at[c, t, pl.ds(g0, GSEG)], gates_sh.at[pl.ds(g0, GSEG)])
        plsc.subcore_barrier()

        _fire_idx(0, 0)
        _fire_idx(1, 1)
        _wait_idx(0)
        _fire_gather(0)

        def _penta(i, carry2):
            for off in range(DEPTH):
                k = DEPTH * i + off
                p = off

                @pl.when(k >= 2)
                def _():
                    _wait_scatter((off - 2) % DEPTH)

                @pl.when(k + 2 < NCHUNK)
                def _():
                    _fire_idx(k + 2, (off + 2) % DEPTH)

                @pl.when(k + 1 < NCHUNK)
                def _():
                    _wait_idx((off + 1) % DEPTH)
                    _fire_gather((off + 1) % DEPTH)

                _wait_gather(p)
                _mult(p)
                _fire_scatter(p)
            return carry2

        lax.fori_loop(0, NCHUNK // DEPTH, _penta, 0)
        _wait_scatter((NCHUNK - 2) % DEPTH)
        _wait_scatter((NCHUNK - 1) % DEPTH)
        plsc.subcore_barrier()

        pltpu.sync_copy(gates_sh.at[pl.ds(n0, NSEG)], ibuf)
        pltpu.sync_copy(gates_sh.at[pl.ds(N + n0, NSEG)], fbuf)
        pltpu.sync_copy(gates_sh.at[pl.ds(2 * N + n0, NSEG)], g2buf)
        pltpu.sync_copy(gates_sh.at[pl.ds(3 * N + n0, NSEG)], obuf)

        def _ew(r, carry2):
            cn = _sigmoid(fbuf[r]) * cbuf[r] + _sigmoid(ibuf[r]) * _tanh(g2buf[r])
            cbuf[r] = cn
            hbuf[r] = _sigmoid(obuf[r]) * _tanh(cn)
            return carry2

        lax.fori_loop(0, NSEG, _ew, 0)
        pltpu.sync_copy(hbuf, h_hbm.at[pl.ds(c * N + n0, NSEG)])
        pltpu.sync_copy(hbuf, out_hbm.at[c, t, pl.ds(n0, NSEG)])
        plsc.subcore_barrier()
        return carry

    lax.fori_loop(0, T, _step, 0)


_sc_call = pl.kernel(
    _sc_body,
    out_type=(jax.ShapeDtypeStruct((NC, T, N, HB), jnp.float32),
              jax.ShapeDtypeStruct((NC * N, HB), jnp.float32)),
    mesh=plsc.VectorSubcoreMesh(core_axis_name="c", subcore_axis_name="s"),
    scratch_types=[
        pltpu.VMEM((2, CH), jnp.int32),
        pltpu.VMEM((2, CH), jnp.int32),
        pltpu.VMEM((2, CH), jnp.int32),
        pltpu.VMEM((2, CH), jnp.int32),
        pltpu.VMEM((2, CH), jnp.int32),
        pltpu.VMEM((CH,), jnp.float32),
        pltpu.VMEM((CH,), jnp.float32),
        pltpu.VMEM((CH,), jnp.float32),
        pltpu.VMEM((CH,), jnp.float32),
        pltpu.VMEM((CH,), jnp.float32),
        pltpu.VMEM((CH, HB), jnp.float32),
        pltpu.VMEM((CH, HB), jnp.float32),
        pltpu.VMEM((CH, HB), jnp.float32),
        pltpu.VMEM((CH, HB), jnp.float32),
        pltpu.VMEM((CH, HB), jnp.float32),
        pltpu.VMEM((NSEG, HB), jnp.float32),
        pltpu.VMEM((NSEG, HB), jnp.float32),
        pltpu.VMEM((NSEG, HB), jnp.float32),
        pltpu.VMEM((NSEG, HB), jnp.float32),
        pltpu.VMEM((NSEG, HB), jnp.float32),
        pltpu.VMEM((NSEG, HB), jnp.float32),
        pltpu.VMEM_SHARED((G, HB), jnp.float32),
        pltpu.SemaphoreType.DMA,
        pltpu.SemaphoreType.DMA,
        pltpu.SemaphoreType.DMA,
    ],
    compiler_params=pltpu.CompilerParams(use_tc_tiling_on_sc=False),
)


def kernel(inputs, W_in, rows, cols, vals, bias):
    xproj = _xproj_call(inputs.reshape(B * T, DIN), W_in, bias)
    xp4 = xproj.reshape(NC, HB, T, G).transpose(0, 2, 3, 1)

    npad = NNZ_PAD - NNZ
    pad_ar = jnp.arange(npad, dtype=jnp.int32)
    rows_p = jnp.concatenate([rows, pad_ar % N]).reshape(NNZ_PAD // CH, 1, CH)
    cols_p = jnp.concatenate([cols, pad_ar % G]).reshape(NNZ_PAD // CH, 1, CH)
    idx_p = jnp.concatenate([rows_p, cols_p], axis=1)
    vals_p = jnp.concatenate([vals, jnp.zeros((npad,), jnp.float32)])

    hs4, _ = _sc_call(xp4, idx_p, vals_p)
    return hs4.transpose(0, 3, 1, 2).reshape(B, T, N)


# speedup vs baseline: 1.2540x; 1.2540x over previous
import functools

import jax
import jax.numpy as jnp
from jax import lax
from jax.experimental import pallas as pl
from jax.experimental.pallas import tpu as pltpu
from jax.experimental.pallas import tpu_sc as plsc

N = 4096
G = 4 * N
NNZ = 671088
DIN = 256
B = 32
T = 16

NC = 2
NS = 16
HB = B // NC
CH = 512
DEPTH = 5
NCHUNK = 85
NNZ_T = CH * NCHUNK
NSTR = 1
NSEG = N // NS
GSEG = G // NS
NNZ_PAD = NS * NNZ_T


def _xproj_body(x_ref, w_ref, b_ref, o_ref):
    o_ref[...] = (
        jnp.dot(x_ref[...], w_ref[...], preferred_element_type=jnp.float32)
        + b_ref[...][None, :]
    )


_xproj_call = pl.pallas_call(
    _xproj_body,
    grid=(G // 512,),
    in_specs=[
        pl.BlockSpec((B * T, DIN), lambda g: (0, 0)),
        pl.BlockSpec((DIN, 512), lambda g: (0, g)),
        pl.BlockSpec((512,), lambda g: (g,)),
    ],
    out_specs=pl.BlockSpec((B * T, 512), lambda g: (0, g)),
    out_shape=jax.ShapeDtypeStruct((B * T, G), jnp.float32),
)


def _splat(v, j):
    idx = jnp.full((16, 1), j, dtype=jnp.int32)
    dnums = lax.GatherDimensionNumbers(
        offset_dims=(), collapsed_slice_dims=(0,), start_index_map=(0,)
    )
    return lax.gather(
        v, idx, dnums, slice_sizes=(1,),
        mode=lax.GatherScatterMode.PROMISE_IN_BOUNDS,
    )


def _sigmoid(x):
    return 1.0 / (1.0 + jnp.exp(-x))


def _tanh(x):
    return 2.0 / (1.0 + jnp.exp(-2.0 * x)) - 1.0


def _sc_body(xp_hbm, idx_hbm, vals_hbm, out_hbm,
             ib0, ib1, ib2, ib3, ib4, vb0, vb1, vb2, vb3, vb4,
             gb0, gb1, gb2, gb3, gb4,
             ibuf, fbuf, g2buf, obuf, cbuf, hbuf,
             h_sh, gates_sh, semI, semG, semS):
    c = lax.axis_index("c")
    s = lax.axis_index("s")
    n0 = s * NSEG
    g0 = s * GSEG
    grp0 = s * (NNZ_T // 128)

    ib = (ib0, ib1, ib2, ib3, ib4)
    vb = (vb0, vb1, vb2, vb3, vb4)
    gb = (gb0, gb1, gb2, gb3, gb4)

    def _fire_idx(k, p):
        pltpu.async_copy(idx_hbm.at[s * NCHUNK + k], ib[p], semI)
        pltpu.async_copy(vals_hbm.at[pl.ds(s * NNZ_T + k * CH, CH)], vb[p], semI)

    def _wait_idx(p):
        pltpu.make_async_copy(idx_hbm.at[0], ib[p], semI).wait()
        pltpu.make_async_copy(vals_hbm.at[pl.ds(0, CH)], vb[p], semI).wait()

    def _fire_gather(p):
        pltpu.async_copy(h_sh.at[ib[p].at[0]], gb[p], semG)

    def _wait_gather(p):
        pltpu.make_async_copy(h_sh.at[ib[p].at[0]], gb[p], semG).wait()

    def _fire_scatter(p):
        pltpu.async_copy(gb[p], gates_sh.at[ib[p].at[1]], semS, add=True)

    def _wait_scatter(p):
        pltpu.make_async_copy(gb[p], gates_sh.at[ib[p].at[1]], semS).wait()

    def _mult(p):
        @plsc.parallel_loop(0, CH // 16, unroll=2)
        def _body(q):
            vv = vb[p][pl.ds(q * 16, 16)]
            base = q * 16
            for j in range(16):
                gb[p][base + j] = gb[p][base + j] * _splat(vv, j)

    def _zero(r, carry):
        hbuf[r] = jnp.zeros((HB,), jnp.float32)
        cbuf[r] = jnp.zeros((HB,), jnp.float32)
        return carry

    lax.fori_loop(0, NSEG, _zero, 0)
    pltpu.sync_copy(hbuf, h_sh.at[pl.ds(n0, NSEG)])
    plsc.subcore_barrier()

    def _step(t, carry):
        pltpu.sync_copy(xp_hbm.at[c, t, pl.ds(g0, GSEG)], gates_sh.at[pl.ds(g0, GSEG)])
        plsc.subcore_barrier()

        _fire_idx(0, 0)
        _fire_idx(1, 1)
        _wait_idx(0)
        _fire_gather(0)

        def _penta(i, carry2):
            for off in range(DEPTH):
                k = DEPTH * i + off
                p = off

                @pl.when(k >= 2)
                def _():
                    _wait_scatter((off - 2) % DEPTH)

                @pl.when(k + 2 < NCHUNK)
                def _():
                    _fire_idx(k + 2, (off + 2) % DEPTH)

                @pl.when(k + 1 < NCHUNK)
                def _():
                    _wait_idx((off + 1) % DEPTH)
                    _fire_gather((off + 1) % DEPTH)

                _wait_gather(p)
                _mult(p)
                _fire_scatter(p)
            return carry2

        lax.fori_loop(0, NCHUNK // DEPTH, _penta, 0)
        _wait_scatter((NCHUNK - 2) % DEPTH)
        _wait_scatter((NCHUNK - 1) % DEPTH)
        plsc.subcore_barrier()

        pltpu.sync_copy(gates_sh.at[pl.ds(n0, NSEG)], ibuf)
        pltpu.sync_copy(gates_sh.at[pl.ds(N + n0, NSEG)], fbuf)
        pltpu.sync_copy(gates_sh.at[pl.ds(2 * N + n0, NSEG)], g2buf)
        pltpu.sync_copy(gates_sh.at[pl.ds(3 * N + n0, NSEG)], obuf)

        def _ew(r, carry2):
            cn = _sigmoid(fbuf[r]) * cbuf[r] + _sigmoid(ibuf[r]) * _tanh(g2buf[r])
            cbuf[r] = cn
            hbuf[r] = _sigmoid(obuf[r]) * _tanh(cn)
            return carry2

        lax.fori_loop(0, NSEG, _ew, 0)
        pltpu.sync_copy(hbuf, h_sh.at[pl.ds(n0, NSEG)])
        pltpu.sync_copy(hbuf, out_hbm.at[c, t, pl.ds(n0, NSEG)])
        plsc.subcore_barrier()
        return carry

    lax.fori_loop(0, T, _step, 0)


_sc_call = pl.kernel(
    _sc_body,
    out_type=jax.ShapeDtypeStruct((NC, T, N, HB), jnp.float32),
    mesh=plsc.VectorSubcoreMesh(core_axis_name="c", subcore_axis_name="s"),
    scratch_types=[
        pltpu.VMEM((2, CH), jnp.int32),
        pltpu.VMEM((2, CH), jnp.int32),
        pltpu.VMEM((2, CH), jnp.int32),
        pltpu.VMEM((2, CH), jnp.int32),
        pltpu.VMEM((2, CH), jnp.int32),
        pltpu.VMEM((CH,), jnp.float32),
        pltpu.VMEM((CH,), jnp.float32),
        pltpu.VMEM((CH,), jnp.float32),
        pltpu.VMEM((CH,), jnp.float32),
        pltpu.VMEM((CH,), jnp.float32),
        pltpu.VMEM((CH, HB), jnp.float32),
        pltpu.VMEM((CH, HB), jnp.float32),
        pltpu.VMEM((CH, HB), jnp.float32),
        pltpu.VMEM((CH, HB), jnp.float32),
        pltpu.VMEM((CH, HB), jnp.float32),
        pltpu.VMEM((NSEG, HB), jnp.float32),
        pltpu.VMEM((NSEG, HB), jnp.float32),
        pltpu.VMEM((NSEG, HB), jnp.float32),
        pltpu.VMEM((NSEG, HB), jnp.float32),
        pltpu.VMEM((NSEG, HB), jnp.float32),
        pltpu.VMEM((NSEG, HB), jnp.float32),
        pltpu.VMEM_SHARED((N, HB), jnp.float32),
        pltpu.VMEM_SHARED((G, HB), jnp.float32),
        pltpu.SemaphoreType.DMA,
        pltpu.SemaphoreType.DMA,
        pltpu.SemaphoreType.DMA,
    ],
    compiler_params=pltpu.CompilerParams(use_tc_tiling_on_sc=False),
)


def kernel(inputs, W_in, rows, cols, vals, bias):
    xproj = _xproj_call(inputs.reshape(B * T, DIN), W_in, bias)
    xp4 = xproj.reshape(NC, HB, T, G).transpose(0, 2, 3, 1)

    npad = NNZ_PAD - NNZ
    pad_ar = jnp.arange(npad, dtype=jnp.int32)
    rows_p = jnp.concatenate([rows, pad_ar % N]).reshape(NNZ_PAD // CH, 1, CH)
    cols_p = jnp.concatenate([cols, pad_ar % G]).reshape(NNZ_PAD // CH, 1, CH)
    idx_p = jnp.concatenate([rows_p, cols_p], axis=1)
    vals_p = jnp.concatenate([vals, jnp.zeros((npad,), jnp.float32)])

    hs4 = _sc_call(xp4, idx_p, vals_p)
    return hs4.transpose(0, 3, 1, 2).reshape(B, T, N)
